# initial kernel scaffold (unmeasured)
import jax
import jax.numpy as jnp
from jax import lax
from jax.experimental import pallas as pl
from jax.experimental.pallas import tpu as pltpu


def kernel(
    x,
):
    def body(*refs):
        pass

    out_shape = jax.ShapeDtypeStruct(..., jnp.float32)
    return pl.pallas_call(body, out_shape=out_shape)(...)


# baseline (device time: 628133 ns/iter reference)
import jax
import jax.numpy as jnp
from jax import lax
from jax.experimental import pallas as pl
from jax.experimental.pallas import tpu as pltpu

N_DEV = 4
STEPS_PER_PASS = 2 * (N_DEV - 1)
N_PASS = 2
N_STEPS = N_PASS * STEPS_PER_PASS


def kernel(x):
    m, n = x.shape
    chunk = m // N_DEV
    half = chunk // 2
    qtr = chunk // 4

    def body(x_hbm, out_hbm,
             send_cw, send_ccw, recv_cw, recv_ccw, loc_cw, loc_ccw,
             send_sem_cw, send_sem_ccw, recv_sems_cw, recv_sems_ccw,
             credit_cw, credit_ccw,
             load_sem_cw, load_sem_ccw, store_sem_cw, store_sem_ccw):
        i = lax.axis_index("i")
        right = (i + 1) % N_DEV
        left = (i + N_DEV - 1) % N_DEV

        barrier = pltpu.get_barrier_semaphore()
        pl.semaphore_signal(barrier, inc=1, device_id=(left,),
                            device_id_type=pl.DeviceIdType.MESH)
        pl.semaphore_signal(barrier, inc=1, device_id=(right,),
                            device_id_type=pl.DeviceIdType.MESH)
        pl.semaphore_wait(barrier, 2)

        def load(c, off, dst, sem):
            cp = pltpu.make_async_copy(
                x_hbm.at[pl.ds(c * chunk + off, qtr), :], dst, sem)
            cp.start()
            return cp

        def store(src, c, off, sem):
            cp = pltpu.make_async_copy(
                src, out_hbm.at[pl.ds(c * chunk + off, qtr), :], sem)
            cp.start()
            return cp

        pending_stores = []
        for s in range(N_STEPS):
            p, t = divmod(s, STEPS_PER_PASS)
            slot = s % 2
            if t == 0:
                for cp in pending_stores:
                    cp.wait()
                pending_stores = []
                lo_cw = load(i, p * qtr, send_cw, load_sem_cw)
                lo_ccw = load(i, half + p * qtr, send_ccw, load_sem_ccw)
                lo_cw.wait()
                lo_ccw.wait()
            if s >= 2:
                pl.semaphore_wait(credit_cw, 1)
                pl.semaphore_wait(credit_ccw, 1)
            r_cw = pltpu.make_async_remote_copy(
                src_ref=send_cw, dst_ref=recv_cw.at[slot],
                send_sem=send_sem_cw, recv_sem=recv_sems_cw.at[slot],
                device_id=(right,), device_id_type=pl.DeviceIdType.MESH)
            r_ccw = pltpu.make_async_remote_copy(
                src_ref=send_ccw, dst_ref=recv_ccw.at[slot],
                send_sem=send_sem_ccw, recv_sem=recv_sems_ccw.at[slot],
                device_id=(left,), device_id_type=pl.DeviceIdType.MESH)
            r_cw.start()
            r_ccw.start()
            if 1 <= s <= N_STEPS - 2:
                pl.semaphore_signal(credit_cw, inc=1, device_id=(left,),
                                    device_id_type=pl.DeviceIdType.MESH)
                pl.semaphore_signal(credit_ccw, inc=1, device_id=(right,),
                                    device_id_type=pl.DeviceIdType.MESH)
            if t < N_DEV - 1:
                lc = load((i - 1 - t) % N_DEV, p * qtr, loc_cw, load_sem_cw)
                lcc = load((i + 1 + t) % N_DEV, half + p * qtr, loc_ccw,
                           load_sem_ccw)
            r_cw.wait()
            r_ccw.wait()
            for cp in pending_stores:
                cp.wait()
            pending_stores = []
            if t < N_DEV - 1:
                lc.wait()
                lcc.wait()
                send_cw[...] = recv_cw[slot] + loc_cw[...]
                send_ccw[...] = recv_ccw[slot] + loc_ccw[...]
                if t == N_DEV - 2:
                    pending_stores.append(
                        store(send_cw, (i + 1) % N_DEV, p * qtr,
                              store_sem_cw))
                    pending_stores.append(
                        store(send_ccw, (i - 1) % N_DEV, half + p * qtr,
                              store_sem_ccw))
            else:
                h = t - (N_DEV - 1)
                send_cw[...] = recv_cw[slot]
                send_ccw[...] = recv_ccw[slot]
                pending_stores.append(
                    store(send_cw, (i - h) % N_DEV, p * qtr, store_sem_cw))
                pending_stores.append(
                    store(send_ccw, (i + h) % N_DEV, half + p * qtr,
                          store_sem_ccw))
        for cp in pending_stores:
            cp.wait()

    return pl.pallas_call(
        body,
        out_shape=jax.ShapeDtypeStruct((m, n), x.dtype),
        in_specs=[pl.BlockSpec(memory_space=pl.ANY)],
        out_specs=pl.BlockSpec(memory_space=pl.ANY),
        scratch_shapes=[
            pltpu.VMEM((qtr, n), x.dtype),
            pltpu.VMEM((qtr, n), x.dtype),
            pltpu.VMEM((2, qtr, n), x.dtype),
            pltpu.VMEM((2, qtr, n), x.dtype),
            pltpu.VMEM((qtr, n), x.dtype),
            pltpu.VMEM((qtr, n), x.dtype),
            pltpu.SemaphoreType.DMA,
            pltpu.SemaphoreType.DMA,
            pltpu.SemaphoreType.DMA((2,)),
            pltpu.SemaphoreType.DMA((2,)),
            pltpu.SemaphoreType.REGULAR,
            pltpu.SemaphoreType.REGULAR,
            pltpu.SemaphoreType.DMA,
            pltpu.SemaphoreType.DMA,
            pltpu.SemaphoreType.DMA,
            pltpu.SemaphoreType.DMA,
        ],
        compiler_params=pltpu.CompilerParams(collective_id=0),
    )(x)


# device time: 626195 ns/iter; 1.0031x vs baseline; 1.0031x over previous
import jax
from jax import lax
from jax.experimental import pallas as pl
from jax.experimental.pallas import tpu as pltpu

N_DEV = 4
N_STEPS = 2 * (N_DEV - 1)


def kernel(x):
    m, n = x.shape
    chunk = m // N_DEV
    half = chunk // 2
    qtr = chunk // 4

    def body(x_hbm, out_hbm,
             send_cw, send_ccw, recv_cw, recv_ccw, loc_cw, loc_ccw,
             send_sem_cw, send_sem_ccw, recv_sems_cw, recv_sems_ccw,
             credit_cw, credit_ccw,
             load_sem_cw, load_sem_ccw, store_sems_cw, store_sems_ccw):
        i = lax.axis_index("i")
        right = (i + 1) % N_DEV
        left = (i + N_DEV - 1) % N_DEV

        def load_q(c, off, dst, sem):
            cp = pltpu.make_async_copy(
                x_hbm.at[pl.ds(c * chunk + off, qtr), :], dst, sem)
            cp.start()
            return cp

        def store_h(src, c, off, sem):
            cp = pltpu.make_async_copy(
                src, out_hbm.at[pl.ds(c * chunk + off, half), :], sem)
            cp.start()
            return cp

        seed_cw = pltpu.make_async_copy(
            x_hbm.at[pl.ds(i * chunk, half), :], send_cw, load_sem_cw)
        seed_ccw = pltpu.make_async_copy(
            x_hbm.at[pl.ds(i * chunk + half, half), :], send_ccw,
            load_sem_ccw)
        seed_cw.start()
        seed_ccw.start()

        barrier = pltpu.get_barrier_semaphore()
        pl.semaphore_signal(barrier, inc=1, device_id=(left,),
                            device_id_type=pl.DeviceIdType.MESH)
        pl.semaphore_signal(barrier, inc=1, device_id=(right,),
                            device_id_type=pl.DeviceIdType.MESH)
        pl.semaphore_wait(barrier, 2)
        seed_cw.wait()
        seed_ccw.wait()

        stores = {}
        for s in range(N_STEPS):
            slot = s % 2
            if s >= 2:
                pl.semaphore_wait(credit_cw, 1)
                pl.semaphore_wait(credit_ccw, 1)
            r_cw = pltpu.make_async_remote_copy(
                src_ref=send_cw, dst_ref=recv_cw.at[slot],
                send_sem=send_sem_cw, recv_sem=recv_sems_cw.at[slot],
                device_id=(right,), device_id_type=pl.DeviceIdType.MESH)
            r_ccw = pltpu.make_async_remote_copy(
                src_ref=send_ccw, dst_ref=recv_ccw.at[slot],
                send_sem=send_sem_ccw, recv_sem=recv_sems_ccw.at[slot],
                device_id=(left,), device_id_type=pl.DeviceIdType.MESH)
            r_cw.start()
            r_ccw.start()
            if 1 <= s <= N_STEPS - 2:
                pl.semaphore_signal(credit_cw, inc=1, device_id=(left,),
                                    device_id_type=pl.DeviceIdType.MESH)
                pl.semaphore_signal(credit_ccw, inc=1, device_id=(right,),
                                    device_id_type=pl.DeviceIdType.MESH)
            if s < N_DEV - 1:
                c_cw = (i - 1 - s) % N_DEV
                c_ccw = (i + 1 + s) % N_DEV
                lc = load_q(c_cw, 0, loc_cw, load_sem_cw)
                lcc = load_q(c_ccw, half, loc_ccw, load_sem_ccw)
            r_cw.wait()
            r_ccw.wait()
            if s < N_DEV - 1:
                lc.wait()
                lcc.wait()
                send_cw[0:qtr, :] = recv_cw[slot, 0:qtr, :] + loc_cw[...]
                send_ccw[0:qtr, :] = recv_ccw[slot, 0:qtr, :] + loc_ccw[...]
                lc2 = load_q(c_cw, qtr, loc_cw, load_sem_cw)
                lcc2 = load_q(c_ccw, half + qtr, loc_ccw, load_sem_ccw)
                lc2.wait()
                lcc2.wait()
                send_cw[qtr:half, :] = recv_cw[slot, qtr:half, :] + loc_cw[...]
                send_ccw[qtr:half, :] = (recv_ccw[slot, qtr:half, :]
                                         + loc_ccw[...])
                if s == N_DEV - 2:
                    stores[(s, "cw")] = store_h(
                        send_cw, (i + 1) % N_DEV, 0, store_sems_cw.at[0])
                    stores[(s, "ccw")] = store_h(
                        send_ccw, (i - 1) % N_DEV, half,
                        store_sems_ccw.at[0])
            else:
                h = s - (N_DEV - 1)
                for key in list(stores):
                    stores.pop(key).wait()
                send_cw[...] = recv_cw[slot]
                send_ccw[...] = recv_ccw[slot]
                stores[(s, "cw")] = store_h(
                    send_cw, (i - h) % N_DEV, 0, store_sems_cw.at[s - 2])
                stores[(s, "ccw")] = store_h(
                    send_ccw, (i + h) % N_DEV, half,
                    store_sems_ccw.at[s - 2])
        for cp in stores.values():
            cp.wait()

    return pl.pallas_call(
        body,
        out_shape=jax.ShapeDtypeStruct((m, n), x.dtype),
        in_specs=[pl.BlockSpec(memory_space=pl.ANY)],
        out_specs=pl.BlockSpec(memory_space=pl.ANY),
        scratch_shapes=[
            pltpu.VMEM((half, n), x.dtype),
            pltpu.VMEM((half, n), x.dtype),
            pltpu.VMEM((2, half, n), x.dtype),
            pltpu.VMEM((2, half, n), x.dtype),
            pltpu.VMEM((qtr, n), x.dtype),
            pltpu.VMEM((qtr, n), x.dtype),
            pltpu.SemaphoreType.DMA,
            pltpu.SemaphoreType.DMA,
            pltpu.SemaphoreType.DMA((2,)),
            pltpu.SemaphoreType.DMA((2,)),
            pltpu.SemaphoreType.REGULAR,
            pltpu.SemaphoreType.REGULAR,
            pltpu.SemaphoreType.DMA,
            pltpu.SemaphoreType.DMA,
            pltpu.SemaphoreType.DMA((4,)),
            pltpu.SemaphoreType.DMA((4,)),
        ],
        compiler_params=pltpu.CompilerParams(
            collective_id=0, vmem_limit_bytes=60 * 1024 * 1024),
    )(x)


# device time: 596597 ns/iter; 1.0529x vs baseline; 1.0496x over previous
import jax
from jax import lax
from jax.experimental import pallas as pl
from jax.experimental.pallas import tpu as pltpu

N_DEV = 4
N_STEPS = 2 * (N_DEV - 1)


def kernel(x):
    m, n = x.shape
    chunk = m // N_DEV
    half = chunk // 2
    qtr = chunk // 4

    def body(x_hbm, out_hbm,
             send_cw, send_ccw, recv_cw, recv_ccw, loc_cw, loc_ccw,
             send_sems_cw, send_sems_ccw, recv_sems_cw, recv_sems_ccw,
             cred_cw0, cred_cw1, cred_ccw0, cred_ccw1,
             load_sem_cw, load_sem_ccw, store_sems_cw, store_sems_ccw):
        i = lax.axis_index("i")
        right = (i + 1) % N_DEV
        left = (i + N_DEV - 1) % N_DEV
        cred_cw = (cred_cw0, cred_cw1)
        cred_ccw = (cred_ccw0, cred_ccw1)

        def load_q(c, off, dst, sem):
            cp = pltpu.make_async_copy(
                x_hbm.at[pl.ds(c * chunk + off, qtr), :], dst, sem)
            cp.start()
            return cp

        def store_q(src, c, off, sem):
            cp = pltpu.make_async_copy(
                src, out_hbm.at[pl.ds(c * chunk + off, qtr), :], sem)
            cp.start()
            return cp

        seed_cw = pltpu.make_async_copy(
            x_hbm.at[pl.ds(i * chunk, half), :], send_cw, load_sem_cw)
        seed_ccw = pltpu.make_async_copy(
            x_hbm.at[pl.ds(i * chunk + half, half), :], send_ccw,
            load_sem_ccw)
        seed_cw.start()
        seed_ccw.start()

        barrier = pltpu.get_barrier_semaphore()
        pl.semaphore_signal(barrier, inc=1, device_id=(left,),
                            device_id_type=pl.DeviceIdType.MESH)
        pl.semaphore_signal(barrier, inc=1, device_id=(right,),
                            device_id_type=pl.DeviceIdType.MESH)
        pl.semaphore_wait(barrier, 2)
        seed_cw.wait()
        seed_ccw.wait()

        def mk(s, j, cw):
            slot = s % 2
            rows = pl.ds(j * qtr, qtr)
            if cw:
                return pltpu.make_async_remote_copy(
                    src_ref=send_cw.at[rows, :],
                    dst_ref=recv_cw.at[slot, rows, :],
                    send_sem=send_sems_cw.at[j],
                    recv_sem=recv_sems_cw.at[slot, j],
                    device_id=(right,),
                    device_id_type=pl.DeviceIdType.MESH)
            return pltpu.make_async_remote_copy(
                src_ref=send_ccw.at[rows, :],
                dst_ref=recv_ccw.at[slot, rows, :],
                send_sem=send_sems_ccw.at[j],
                recv_sem=recv_sems_ccw.at[slot, j],
                device_id=(left,),
                device_id_type=pl.DeviceIdType.MESH)

        rdmas = {}
        for j in (0, 1):
            r1, r2 = mk(0, j, True), mk(0, j, False)
            r1.start()
            r2.start()
            rdmas[("cw", j)] = r1
            rdmas[("ccw", j)] = r2
        lpend = {
            "cw": load_q((i - 1) % N_DEV, 0, loc_cw, load_sem_cw),
            "ccw": load_q((i + 1) % N_DEV, half, loc_ccw, load_sem_ccw),
        }

        stores = {}
        for s in range(N_STEPS):
            slot = s % 2
            for j in (0, 1):
                lo, hi = j * qtr, (j + 1) * qtr
                rdmas.pop(("cw", j)).wait()
                rdmas.pop(("ccw", j)).wait()
                for d in ("cw", "ccw"):
                    cp = stores.pop((d, j), None)
                    if cp is not None:
                        cp.wait()
                if s < N_DEV - 1:
                    lpend["cw"].wait()
                    lpend["ccw"].wait()
                    send_cw[lo:hi, :] = recv_cw[slot, lo:hi, :] + loc_cw[...]
                    send_ccw[lo:hi, :] = (recv_ccw[slot, lo:hi, :]
                                          + loc_ccw[...])
                    ns, nj = (s, 1) if j == 0 else (s + 1, 0)
                    if ns < N_DEV - 1:
                        lpend["cw"] = load_q(
                            (i - 1 - ns) % N_DEV, nj * qtr, loc_cw,
                            load_sem_cw)
                        lpend["ccw"] = load_q(
                            (i + 1 + ns) % N_DEV, half + nj * qtr, loc_ccw,
                            load_sem_ccw)
                    if s == N_DEV - 2:
                        stores[("cw", j)] = store_q(
                            send_cw.at[pl.ds(lo, qtr), :], (i + 1) % N_DEV,
                            lo, store_sems_cw.at[j])
                        stores[("ccw", j)] = store_q(
                            send_ccw.at[pl.ds(lo, qtr), :], (i - 1) % N_DEV,
                            half + lo, store_sems_ccw.at[j])
                else:
                    h = s - (N_DEV - 1)
                    send_cw[lo:hi, :] = recv_cw[slot, lo:hi, :]
                    send_ccw[lo:hi, :] = recv_ccw[slot, lo:hi, :]
                    stores[("cw", j)] = store_q(
                        send_cw.at[pl.ds(lo, qtr), :], (i - h) % N_DEV, lo,
                        store_sems_cw.at[j])
                    stores[("ccw", j)] = store_q(
                        send_ccw.at[pl.ds(lo, qtr), :], (i + h) % N_DEV,
                        half + lo, store_sems_ccw.at[j])
                if s < N_STEPS - 1:
                    if s + 1 >= 2:
                        pl.semaphore_wait(cred_cw[j], 1)
                        pl.semaphore_wait(cred_ccw[j], 1)
                    r1, r2 = mk(s + 1, j, True), mk(s + 1, j, False)
                    r1.start()
                    r2.start()
                    rdmas[("cw", j)] = r1
                    rdmas[("ccw", j)] = r2
                    if s + 1 <= N_STEPS - 2:
                        pl.semaphore_signal(
                            cred_cw[j], inc=1, device_id=(left,),
                            device_id_type=pl.DeviceIdType.MESH)
                        pl.semaphore_signal(
                            cred_ccw[j], inc=1, device_id=(right,),
                            device_id_type=pl.DeviceIdType.MESH)
        for cp in stores.values():
            cp.wait()

    return pl.pallas_call(
        body,
        out_shape=jax.ShapeDtypeStruct((m, n), x.dtype),
        in_specs=[pl.BlockSpec(memory_space=pl.ANY)],
        out_specs=pl.BlockSpec(memory_space=pl.ANY),
        scratch_shapes=[
            pltpu.VMEM((half, n), x.dtype),
            pltpu.VMEM((half, n), x.dtype),
            pltpu.VMEM((2, half, n), x.dtype),
            pltpu.VMEM((2, half, n), x.dtype),
            pltpu.VMEM((qtr, n), x.dtype),
            pltpu.VMEM((qtr, n), x.dtype),
            pltpu.SemaphoreType.DMA((2,)),
            pltpu.SemaphoreType.DMA((2,)),
            pltpu.SemaphoreType.DMA((2, 2)),
            pltpu.SemaphoreType.DMA((2, 2)),
            pltpu.SemaphoreType.REGULAR,
            pltpu.SemaphoreType.REGULAR,
            pltpu.SemaphoreType.REGULAR,
            pltpu.SemaphoreType.REGULAR,
            pltpu.SemaphoreType.DMA,
            pltpu.SemaphoreType.DMA,
            pltpu.SemaphoreType.DMA((2,)),
            pltpu.SemaphoreType.DMA((2,)),
        ],
        compiler_params=pltpu.CompilerParams(
            collective_id=0, vmem_limit_bytes=60 * 1024 * 1024),
    )(x)


# device time: 596160 ns/iter; 1.0536x vs baseline; 1.0007x over previous
import jax
from jax import lax
from jax.experimental import pallas as pl
from jax.experimental.pallas import tpu as pltpu

N_DEV = 4
N_STEPS = 2 * (N_DEV - 1)


def kernel(x):
    m, n = x.shape
    chunk = m // N_DEV
    half = chunk // 2
    qtr = chunk // 4

    def body(x_hbm, out_hbm,
             send_cw, send_ccw, recv_cw, recv_ccw, loc_cw, loc_ccw,
             send_sems_cw, send_sems_ccw, recv_sems_cw, recv_sems_ccw,
             cred_cw0, cred_cw1, cred_ccw0, cred_ccw1,
             load_sem_cw, load_sem_ccw, store_sems_cw, store_sems_ccw):
        i = lax.axis_index("i")
        right = (i + 1) % N_DEV
        left = (i + N_DEV - 1) % N_DEV
        cred_cw = (cred_cw0, cred_cw1)
        cred_ccw = (cred_ccw0, cred_ccw1)

        def load_q(c, off, dst, sem):
            cp = pltpu.make_async_copy(
                x_hbm.at[pl.ds(c * chunk + off, qtr), :], dst, sem)
            cp.start()
            return cp

        def store_q(src, c, off, sem):
            cp = pltpu.make_async_copy(
                src, out_hbm.at[pl.ds(c * chunk + off, qtr), :], sem)
            cp.start()
            return cp

        seed_cw = pltpu.make_async_copy(
            x_hbm.at[pl.ds(i * chunk, half), :], send_cw, load_sem_cw)
        seed_ccw = pltpu.make_async_copy(
            x_hbm.at[pl.ds(i * chunk + half, half), :], send_ccw,
            load_sem_ccw)
        seed_cw.start()
        seed_ccw.start()

        barrier = pltpu.get_barrier_semaphore()
        pl.semaphore_signal(barrier, inc=1, device_id=(left,),
                            device_id_type=pl.DeviceIdType.MESH)
        pl.semaphore_signal(barrier, inc=1, device_id=(right,),
                            device_id_type=pl.DeviceIdType.MESH)
        pl.semaphore_wait(barrier, 2)
        seed_cw.wait()
        seed_ccw.wait()

        def mk(s, j, cw):
            slot = s % 2
            rows = pl.ds(j * qtr, qtr)
            if cw:
                return pltpu.make_async_remote_copy(
                    src_ref=send_cw.at[rows, :],
                    dst_ref=recv_cw.at[slot, rows, :],
                    send_sem=send_sems_cw.at[j],
                    recv_sem=recv_sems_cw.at[slot, j],
                    device_id=(right,),
                    device_id_type=pl.DeviceIdType.MESH)
            return pltpu.make_async_remote_copy(
                src_ref=send_ccw.at[rows, :],
                dst_ref=recv_ccw.at[slot, rows, :],
                send_sem=send_sems_ccw.at[j],
                recv_sem=recv_sems_ccw.at[slot, j],
                device_id=(left,),
                device_id_type=pl.DeviceIdType.MESH)

        rdmas = {}
        for j in (0, 1):
            r1, r2 = mk(0, j, True), mk(0, j, False)
            r1.start()
            r2.start()
            rdmas[("cw", j)] = r1
            rdmas[("ccw", j)] = r2
        lpend = {
            "cw": load_q((i - 1) % N_DEV, 0, loc_cw, load_sem_cw),
            "ccw": load_q((i + 1) % N_DEV, half, loc_ccw, load_sem_ccw),
        }

        stores = {}
        for s in range(N_STEPS):
            slot = s % 2
            for j in (0, 1):
                lo, hi = j * qtr, (j + 1) * qtr
                rdmas.pop(("cw", j)).wait()
                rdmas.pop(("ccw", j)).wait()
                for d in ("cw", "ccw"):
                    cp = stores.pop((d, j), None)
                    if cp is not None:
                        cp.wait()
                if s < N_DEV - 1:
                    lpend["cw"].wait()
                    lpend["ccw"].wait()
                    send_cw[lo:hi, :] = recv_cw[slot, lo:hi, :] + loc_cw[...]
                    send_ccw[lo:hi, :] = (recv_ccw[slot, lo:hi, :]
                                          + loc_ccw[...])
                    ns, nj = (s, 1) if j == 0 else (s + 1, 0)
                    if ns < N_DEV - 1:
                        lpend["cw"] = load_q(
                            (i - 1 - ns) % N_DEV, nj * qtr, loc_cw,
                            load_sem_cw)
                        lpend["ccw"] = load_q(
                            (i + 1 + ns) % N_DEV, half + nj * qtr, loc_ccw,
                            load_sem_ccw)
                    if s == N_DEV - 2:
                        stores[("cw", j)] = store_q(
                            send_cw.at[pl.ds(lo, qtr), :], (i + 1) % N_DEV,
                            lo, store_sems_cw.at[j])
                        stores[("ccw", j)] = store_q(
                            send_ccw.at[pl.ds(lo, qtr), :], (i - 1) % N_DEV,
                            half + lo, store_sems_ccw.at[j])
                else:
                    h = s - (N_DEV - 1)
                    if s < N_STEPS - 1:
                        send_cw[lo:hi, :] = recv_cw[slot, lo:hi, :]
                        send_ccw[lo:hi, :] = recv_ccw[slot, lo:hi, :]
                        src_cw = send_cw.at[pl.ds(lo, qtr), :]
                        src_ccw = send_ccw.at[pl.ds(lo, qtr), :]
                    else:
                        src_cw = recv_cw.at[slot, pl.ds(lo, qtr), :]
                        src_ccw = recv_ccw.at[slot, pl.ds(lo, qtr), :]
                    stores[("cw", j)] = store_q(
                        src_cw, (i - h) % N_DEV, lo, store_sems_cw.at[j])
                    stores[("ccw", j)] = store_q(
                        src_ccw, (i + h) % N_DEV, half + lo,
                        store_sems_ccw.at[j])
                if s < N_STEPS - 1:
                    if s + 1 >= 2:
                        pl.semaphore_wait(cred_cw[j], 1)
                        pl.semaphore_wait(cred_ccw[j], 1)
                    r1, r2 = mk(s + 1, j, True), mk(s + 1, j, False)
                    r1.start()
                    r2.start()
                    rdmas[("cw", j)] = r1
                    rdmas[("ccw", j)] = r2
                    if s + 1 <= N_STEPS - 2:
                        pl.semaphore_signal(
                            cred_cw[j], inc=1, device_id=(left,),
                            device_id_type=pl.DeviceIdType.MESH)
                        pl.semaphore_signal(
                            cred_ccw[j], inc=1, device_id=(right,),
                            device_id_type=pl.DeviceIdType.MESH)
        for cp in stores.values():
            cp.wait()

    return pl.pallas_call(
        body,
        out_shape=jax.ShapeDtypeStruct((m, n), x.dtype),
        in_specs=[pl.BlockSpec(memory_space=pl.ANY)],
        out_specs=pl.BlockSpec(memory_space=pl.ANY),
        scratch_shapes=[
            pltpu.VMEM((half, n), x.dtype),
            pltpu.VMEM((half, n), x.dtype),
            pltpu.VMEM((2, half, n), x.dtype),
            pltpu.VMEM((2, half, n), x.dtype),
            pltpu.VMEM((qtr, n), x.dtype),
            pltpu.VMEM((qtr, n), x.dtype),
            pltpu.SemaphoreType.DMA((2,)),
            pltpu.SemaphoreType.DMA((2,)),
            pltpu.SemaphoreType.DMA((2, 2)),
            pltpu.SemaphoreType.DMA((2, 2)),
            pltpu.SemaphoreType.REGULAR,
            pltpu.SemaphoreType.REGULAR,
            pltpu.SemaphoreType.REGULAR,
            pltpu.SemaphoreType.REGULAR,
            pltpu.SemaphoreType.DMA,
            pltpu.SemaphoreType.DMA,
            pltpu.SemaphoreType.DMA((2,)),
            pltpu.SemaphoreType.DMA((2,)),
        ],
        compiler_params=pltpu.CompilerParams(
            collective_id=0, vmem_limit_bytes=60 * 1024 * 1024),
    )(x)
